# baseline (speedup 1.0000x reference)
import jax
import jax.numpy as jnp
from jax import lax
from jax.experimental import pallas as pl
from jax.experimental.pallas import tpu as pltpu
from jax.experimental.pallas import tpu_sc as plsc

ENTITY_TOTAL = 1000000
RELATION_TOTAL = 1000
EMB = 64
B = 16384

NC = 2
NS = 16
NW = NC * NS
LANES = 16
KREGS = EMB // LANES

PER_W = B // NW
SUB = 64
NCHUNK = PER_W // SUB
GROUPS = SUB // LANES


def _project(j, off, e_ref, p_ref, offr, rp_ref):
    ev = [e_ref[j, pl.ds(off + k * LANES, LANES)] for k in range(KREGS)]
    pv = [p_ref[j, pl.ds(off + k * LANES, LANES)] for k in range(KREGS)]
    rpv = [rp_ref[j, pl.ds(offr + k * LANES, LANES)] for k in range(KREGS)]
    prod = ev[0] * pv[0]
    for k in range(1, KREGS):
        prod = prod + ev[k] * pv[k]
    dot = jnp.sum(prod)
    return [ev[k] + dot * rpv[k] for k in range(KREGS)]


def _side(j, g, jj, ohv, otv, orv, he, te, re, hp, tp, rp, ho, to):
    offh = ohv[jj]
    offt = otv[jj]
    offr = orv[jj]
    hn = _project(j, offh, he, hp, offr, rp)
    tn = _project(j, offt, te, tp, offr, rp)
    orow = g * (LANES // 2) + jj // 2
    ocol = (jj % 2) * EMB
    s = None
    for k in range(KREGS):
        ho[orow, pl.ds(ocol + k * LANES, LANES)] = hn[k]
        to[orow, pl.ds(ocol + k * LANES, LANES)] = tn[k]
        rv = re[j, pl.ds(offr + k * LANES, LANES)]
        term = jnp.abs(hn[k] + rv - tn[k])
        s = term if s is None else s + term
    return jnp.sum(s)


def _body(pos_h_hbm, pos_t_hbm, pos_r_hbm, neg_h_hbm, neg_t_hbm, neg_r_hbm,
          ent_hbm, rel_hbm, entp_hbm, relp_hbm,
          pos_out, neg_out, phe_out, pte_out, nhe_out, nte_out,
          iph, ipt, ipr, inh, int_, inr,
          sph, spt, spr, snh, snt, snr,
          phe, pte, pre, php, ptp, prp,
          nhe, nte, nre, nhp, ntp, nrp,
          pho, pto, nho, nto,
          psc, nsc, sem):
    wid = lax.axis_index("s") * NC + lax.axis_index("c")
    base = wid * PER_W
    lane = lax.iota(jnp.int32, LANES)

    def chunk(c, carry):
        off = pl.multiple_of(base + c * SUB, SUB)
        pltpu.sync_copy(pos_h_hbm.at[pl.ds(off, SUB)], iph)
        pltpu.sync_copy(pos_t_hbm.at[pl.ds(off, SUB)], ipt)
        pltpu.sync_copy(pos_r_hbm.at[pl.ds(off, SUB)], ipr)
        pltpu.sync_copy(neg_h_hbm.at[pl.ds(off, SUB)], inh)
        pltpu.sync_copy(neg_t_hbm.at[pl.ds(off, SUB)], int_)
        pltpu.sync_copy(neg_r_hbm.at[pl.ds(off, SUB)], inr)
        for raw, sh in ((iph, sph), (ipt, spt), (ipr, spr),
                        (inh, snh), (int_, snt), (inr, snr)):
            for q in range(SUB // LANES):
                sl = pl.ds(q * LANES, LANES)
                sh[sl] = lax.shift_right_logical(raw[sl], 1)
        copies = [
            pltpu.async_copy(ent_hbm.at[sph], phe, sem),
            pltpu.async_copy(ent_hbm.at[spt], pte, sem),
            pltpu.async_copy(rel_hbm.at[spr], pre, sem),
            pltpu.async_copy(entp_hbm.at[sph], php, sem),
            pltpu.async_copy(entp_hbm.at[spt], ptp, sem),
            pltpu.async_copy(relp_hbm.at[spr], prp, sem),
            pltpu.async_copy(ent_hbm.at[snh], nhe, sem),
            pltpu.async_copy(ent_hbm.at[snt], nte, sem),
            pltpu.async_copy(rel_hbm.at[snr], nre, sem),
            pltpu.async_copy(entp_hbm.at[snh], nhp, sem),
            pltpu.async_copy(entp_hbm.at[snt], ntp, sem),
            pltpu.async_copy(relp_hbm.at[snr], nrp, sem),
        ]
        for cp in copies:
            cp.wait()

        def group(g, carry2):
            pacc = jnp.zeros((LANES,), jnp.float32)
            nacc = jnp.zeros((LANES,), jnp.float32)
            gsl = pl.ds(g * LANES, LANES)
            poh = (iph[gsl] & 1) * EMB
            pot = (ipt[gsl] & 1) * EMB
            por = (ipr[gsl] & 1) * EMB
            noh = (inh[gsl] & 1) * EMB
            not_ = (int_[gsl] & 1) * EMB
            nor = (inr[gsl] & 1) * EMB
            for jj in range(LANES):
                j = g * LANES + jj
                ps = _side(j, g, jj, poh, pot, por,
                           phe, pte, pre, php, ptp, prp, pho, pto)
                ns = _side(j, g, jj, noh, not_, nor,
                           nhe, nte, nre, nhp, ntp, nrp, nho, nto)
                m = lane == jj
                pacc = jnp.where(m, ps, pacc)
                nacc = jnp.where(m, ns, nacc)
            psc[pl.ds(g * LANES, LANES)] = pacc
            nsc[pl.ds(g * LANES, LANES)] = nacc
            return carry2

        lax.fori_loop(0, GROUPS, group, 0)

        half = pl.ds(pl.multiple_of(off // 2, SUB // 2), SUB // 2)
        pltpu.sync_copy(pho, phe_out.at[half])
        pltpu.sync_copy(pto, pte_out.at[half])
        pltpu.sync_copy(nho, nhe_out.at[half])
        pltpu.sync_copy(nto, nte_out.at[half])
        pltpu.sync_copy(psc, pos_out.at[pl.ds(off, SUB)])
        pltpu.sync_copy(nsc, neg_out.at[pl.ds(off, SUB)])
        return carry

    lax.fori_loop(0, NCHUNK, chunk, 0)


@jax.jit
def kernel(pos_h, pos_t, pos_r, neg_h, neg_t, neg_r,
           ent_emb, rel_emb, ent_proj_emb, rel_proj_emb):
    f32 = jnp.float32
    run = pl.kernel(
        _body,
        out_type=(
            jax.ShapeDtypeStruct((B,), f32),
            jax.ShapeDtypeStruct((B,), f32),
            jax.ShapeDtypeStruct((B // 2, 2 * EMB), f32),
            jax.ShapeDtypeStruct((B // 2, 2 * EMB), f32),
            jax.ShapeDtypeStruct((B // 2, 2 * EMB), f32),
            jax.ShapeDtypeStruct((B // 2, 2 * EMB), f32),
        ),
        mesh=plsc.VectorSubcoreMesh(
            core_axis_name="c", subcore_axis_name="s",
            num_cores=NC, num_subcores=NS),
        compiler_params=pltpu.CompilerParams(
            needs_layout_passes=False, use_tc_tiling_on_sc=True),
        scratch_types=(
            [pltpu.VMEM((SUB,), jnp.int32)] * 12
            + [pltpu.VMEM((SUB, 2 * EMB), f32)] * 12
            + [pltpu.VMEM((SUB // 2, 2 * EMB), f32)] * 4
            + [pltpu.VMEM((SUB,), f32)] * 2
            + [pltpu.SemaphoreType.DMA]
        ),
    )
    pos, neg, phe, pte, nhe, nte = run(
        pos_h, pos_t, pos_r, neg_h, neg_t, neg_r,
        jnp.reshape(ent_emb, (ENTITY_TOTAL // 2, 2 * EMB)),
        jnp.reshape(rel_emb, (RELATION_TOTAL // 2, 2 * EMB)),
        jnp.reshape(ent_proj_emb, (ENTITY_TOTAL // 2, 2 * EMB)),
        jnp.reshape(rel_proj_emb, (RELATION_TOTAL // 2, 2 * EMB)))
    rs = lambda x: jnp.reshape(x, (B, EMB))
    return (pos, neg, rs(phe), rs(pte), rs(nhe), rs(nte))


# speedup vs baseline: 1.2992x; 1.2992x over previous
import jax
import jax.numpy as jnp
from jax import lax
from jax.experimental import pallas as pl
from jax.experimental.pallas import tpu as pltpu
from jax.experimental.pallas import tpu_sc as plsc

ENTITY_TOTAL = 1000000
RELATION_TOTAL = 1000
EMB = 64
B = 16384

NC = 2
NS = 16
NW = NC * NS
LANES = 16
KREGS = EMB // LANES

NHITS = 4 * B
HPW = NHITS // NW
ASUB = 64
ACHUNKS = HPW // ASUB
BLKW = 128

PER_W = B // NW
SUB = 128
NCHUNK = PER_W // SUB
GROUPS = SUB // LANES


TAILSTART = (ENTITY_TOTAL // BLKW) * BLKW


def _scan_body(entT_hbm, entpT_hbm, se_hbm, taile_hbm, tailp_hbm,
               rawe_out, rawp_out,
               se_v, blk_e, blk_p, tl_e, tl_p, re_st, rp_st):
    wid = lax.axis_index("s") * NC + lax.axis_index("c")
    base = pl.multiple_of(wid * HPW, HPW)
    pltpu.sync_copy(se_hbm.at[pl.ds(base, HPW)], se_v)
    pltpu.sync_copy(taile_hbm, tl_e)
    pltpu.sync_copy(tailp_hbm, tl_p)
    rowv = lax.iota(jnp.int32, LANES)

    def chunk(m, cur):
        for g in range(ASUB // LANES):
            ev = se_v[pl.ds(m * ASUB + g * LANES, LANES)]
            for jj in range(LANES):
                e = ev[jj]
                tail = e >= TAILSTART
                c = lax.shift_right_logical(e, 7)

                @pl.when(jnp.logical_and(c != cur, jnp.logical_not(tail)))
                def _():
                    co = pl.multiple_of(c * BLKW, BLKW)
                    pltpu.sync_copy(entT_hbm.at[:, pl.ds(co, BLKW)], blk_e)
                    pltpu.sync_copy(entpT_hbm.at[:, pl.ds(co, BLKW)], blk_p)

                cur = jnp.where(tail, cur, c)
                r = g * LANES + jj

                @pl.when(jnp.logical_not(tail))
                def _():
                    l = jnp.broadcast_to(e & (BLKW - 1), (LANES,))
                    for k in range(KREGS):
                        rows = rowv + k * LANES
                        re_st[r, pl.ds(k * LANES, LANES)] = plsc.load_gather(
                            blk_e, [rows, l])
                        rp_st[r, pl.ds(k * LANES, LANES)] = plsc.load_gather(
                            blk_p, [rows, l])

                @pl.when(tail)
                def _():
                    row = e - TAILSTART
                    for k in range(KREGS):
                        ksl = pl.ds(k * LANES, LANES)
                        re_st[r, ksl] = tl_e[row, ksl]
                        rp_st[r, ksl] = tl_p[row, ksl]
        dst = pl.ds(pl.multiple_of(base + m * ASUB, ASUB), ASUB)
        pltpu.sync_copy(re_st, rawe_out.at[dst])
        pltpu.sync_copy(rp_st, rawp_out.at[dst])
        return cur

    lax.fori_loop(0, ACHUNKS, chunk, jnp.int32(-1))


def _project(j, e_ref, p_ref, rp_ref):
    ev = [e_ref[j, pl.ds(k * LANES, LANES)] for k in range(KREGS)]
    pv = [p_ref[j, pl.ds(k * LANES, LANES)] for k in range(KREGS)]
    rpv = [rp_ref[j, pl.ds(k * LANES, LANES)] for k in range(KREGS)]
    prod = ev[0] * pv[0]
    for k in range(1, KREGS):
        prod = prod + ev[k] * pv[k]
    dot = jnp.sum(prod)
    return [ev[k] + dot * rpv[k] for k in range(KREGS)]


def _side(j, h_ref, t_ref, r_ref, hp_ref, tp_ref, rp_ref):
    hn = _project(j, h_ref, hp_ref, rp_ref)
    tn = _project(j, t_ref, tp_ref, rp_ref)
    s = None
    for k in range(KREGS):
        rv = r_ref[j, pl.ds(k * LANES, LANES)]
        term = jnp.abs(hn[k] + rv - tn[k])
        s = term if s is None else s + term
    for k in range(KREGS):
        h_ref[j, pl.ds(k * LANES, LANES)] = hn[k]
        t_ref[j, pl.ds(k * LANES, LANES)] = tn[k]
    return jnp.sum(s)


def _main_body(iperm_hbm, pos_r_hbm, neg_r_hbm, rawe_hbm, rawp_hbm,
               rel_hbm, relp_hbm,
               pos_out, neg_out, phe_out, pte_out, nhe_out, nte_out,
               iph, ipt, ipr, inh, int_, inr,
               phe, pte, pre, php, ptp, prp,
               nhe, nte, nre, nhp, ntp, nrp,
               psc, nsc, sem):
    wid = lax.axis_index("s") * NC + lax.axis_index("c")
    base = wid * PER_W
    lane = lax.iota(jnp.int32, LANES)

    def chunk(c, carry):
        off = pl.multiple_of(base + c * SUB, SUB)
        sl = pl.ds(off, SUB)
        pltpu.sync_copy(iperm_hbm.at[pl.ds(0 * B + off, SUB)], iph)
        pltpu.sync_copy(iperm_hbm.at[pl.ds(1 * B + off, SUB)], ipt)
        pltpu.sync_copy(iperm_hbm.at[pl.ds(2 * B + off, SUB)], inh)
        pltpu.sync_copy(iperm_hbm.at[pl.ds(3 * B + off, SUB)], int_)
        pltpu.sync_copy(pos_r_hbm.at[sl], ipr)
        pltpu.sync_copy(neg_r_hbm.at[sl], inr)
        copies = [
            pltpu.async_copy(rawe_hbm.at[iph], phe, sem),
            pltpu.async_copy(rawe_hbm.at[ipt], pte, sem),
            pltpu.async_copy(rel_hbm.at[ipr], pre, sem),
            pltpu.async_copy(rawp_hbm.at[iph], php, sem),
            pltpu.async_copy(rawp_hbm.at[ipt], ptp, sem),
            pltpu.async_copy(relp_hbm.at[ipr], prp, sem),
            pltpu.async_copy(rawe_hbm.at[inh], nhe, sem),
            pltpu.async_copy(rawe_hbm.at[int_], nte, sem),
            pltpu.async_copy(rel_hbm.at[inr], nre, sem),
            pltpu.async_copy(rawp_hbm.at[inh], nhp, sem),
            pltpu.async_copy(rawp_hbm.at[int_], ntp, sem),
            pltpu.async_copy(relp_hbm.at[inr], nrp, sem),
        ]
        for cp in copies:
            cp.wait()

        def group(g, carry2):
            pacc = jnp.zeros((LANES,), jnp.float32)
            nacc = jnp.zeros((LANES,), jnp.float32)
            for jj in range(LANES):
                j = g * LANES + jj
                ps = _side(j, phe, pte, pre, php, ptp, prp)
                ns = _side(j, nhe, nte, nre, nhp, ntp, nrp)
                m = lane == jj
                pacc = jnp.where(m, ps, pacc)
                nacc = jnp.where(m, ns, nacc)
            psc[pl.ds(g * LANES, LANES)] = pacc
            nsc[pl.ds(g * LANES, LANES)] = nacc
            return carry2

        lax.fori_loop(0, GROUPS, group, 0)

        pltpu.sync_copy(phe, phe_out.at[sl])
        pltpu.sync_copy(pte, pte_out.at[sl])
        pltpu.sync_copy(nhe, nhe_out.at[sl])
        pltpu.sync_copy(nte, nte_out.at[sl])
        pltpu.sync_copy(psc, pos_out.at[sl])
        pltpu.sync_copy(nsc, neg_out.at[sl])
        return carry

    lax.fori_loop(0, NCHUNK, chunk, 0)


@jax.jit
def kernel(pos_h, pos_t, pos_r, neg_h, neg_t, neg_r,
           ent_emb, rel_emb, ent_proj_emb, rel_proj_emb):
    f32 = jnp.float32
    i32 = jnp.int32
    mesh = plsc.VectorSubcoreMesh(
        core_axis_name="c", subcore_axis_name="s",
        num_cores=NC, num_subcores=NS)

    hits = jnp.concatenate([pos_h, pos_t, neg_h, neg_t])
    order = jnp.argsort(hits).astype(i32)
    sorted_e = jnp.take(hits, order)
    iperm = jnp.zeros((NHITS,), i32).at[order].set(
        jnp.arange(NHITS, dtype=i32))

    scan = pl.kernel(
        _scan_body,
        out_type=(
            jax.ShapeDtypeStruct((NHITS, EMB), f32),
            jax.ShapeDtypeStruct((NHITS, EMB), f32),
        ),
        mesh=mesh,
        compiler_params=pltpu.CompilerParams(
            needs_layout_passes=False, use_tc_tiling_on_sc=True),
        scratch_types=(
            pltpu.VMEM((HPW,), i32),
            pltpu.VMEM((EMB, BLKW), f32),
            pltpu.VMEM((EMB, BLKW), f32),
            pltpu.VMEM((ENTITY_TOTAL - TAILSTART, EMB), f32),
            pltpu.VMEM((ENTITY_TOTAL - TAILSTART, EMB), f32),
            pltpu.VMEM((ASUB, EMB), f32),
            pltpu.VMEM((ASUB, EMB), f32),
        ),
    )
    raw_e, raw_p = scan(ent_emb.T, ent_proj_emb.T, sorted_e,
                        ent_emb[TAILSTART:], ent_proj_emb[TAILSTART:])

    main = pl.kernel(
        _main_body,
        out_type=(
            jax.ShapeDtypeStruct((B,), f32),
            jax.ShapeDtypeStruct((B,), f32),
            jax.ShapeDtypeStruct((B, EMB), f32),
            jax.ShapeDtypeStruct((B, EMB), f32),
            jax.ShapeDtypeStruct((B, EMB), f32),
            jax.ShapeDtypeStruct((B, EMB), f32),
        ),
        mesh=mesh,
        compiler_params=pltpu.CompilerParams(
            needs_layout_passes=False, use_tc_tiling_on_sc=False),
        scratch_types=(
            [pltpu.VMEM((SUB,), i32)] * 6
            + [pltpu.VMEM((SUB, EMB), f32)] * 12
            + [pltpu.VMEM((SUB,), f32)] * 2
            + [pltpu.SemaphoreType.DMA]
        ),
    )
    return main(iperm, pos_r, neg_r, raw_e, raw_p, rel_emb, rel_proj_emb)
